# baseline (device time: 176459 ns/iter reference)
import jax
import jax.numpy as jnp
from jax import lax
from jax.experimental import pallas as pl
from jax.experimental.pallas import tpu as pltpu

N_DEV = 4


def kernel(x, w_mat, scale_x, scale_w):
    if x.dtype != jnp.float8_e4m3fn:
        x = x.astype(jnp.float8_e4m3fn)
    if w_mat.dtype != jnp.float8_e4m3fn:
        w_mat = w_mat.astype(jnp.float8_e4m3fn)
    scale = (scale_x.astype(jnp.float32) * scale_w.astype(jnp.float32)).reshape((1,))

    m, k = x.shape
    _, n = w_mat.shape
    mh = m // 2
    nh = n // 2

    def body(x_ref, w_ref, scale_ref, out_ref,
             xt_com, xb_com, wl_com, wr_com,
             xt_send, xt_recv, xb_send, xb_recv,
             wl_send, wl_recv, wr_send, wr_recv):
        my = lax.axis_index("i")
        left = lax.rem(my + N_DEV - 1, N_DEV)
        right = lax.rem(my + 1, N_DEV)

        barrier_sem = pltpu.get_barrier_semaphore()
        for nbr in (left, right):
            pl.semaphore_signal(
                barrier_sem, inc=1,
                device_id=(nbr,), device_id_type=pl.DeviceIdType.MESH,
            )
        pl.semaphore_wait(barrier_sem, 2)

        def rdma(src, dst, ssem, rsem, dev):
            return pltpu.make_async_remote_copy(
                src_ref=src, dst_ref=dst, send_sem=ssem, recv_sem=rsem,
                device_id=(dev,), device_id_type=pl.DeviceIdType.MESH,
            )

        def dot32(a, b):
            return jnp.dot(a, b, preferred_element_type=jnp.float32)

        TOP, BOT = pl.ds(0, mh), pl.ds(mh, mh)
        LFT, RGT = pl.ds(0, nh), pl.ds(nh, nh)

        for h in range(N_DEV - 1):
            if h == 0:
                xt_src = x_ref.at[TOP, :]
                xb_src = x_ref.at[BOT, :]
                wl_src = w_ref.at[:, LFT]
                wr_src = w_ref.at[:, RGT]
            else:
                xt_src = xt_com.at[h - 1]
                xb_src = xb_com.at[h - 1]
                wl_src = wl_com.at[h - 1]
                wr_src = wr_com.at[h - 1]
            hops = [
                rdma(xt_src, xt_com.at[h], xt_send.at[h], xt_recv.at[h], right),
                rdma(wl_src, wl_com.at[h], wl_send.at[h], wl_recv.at[h], right),
                rdma(xb_src, xb_com.at[h], xb_send.at[h], xb_recv.at[h], left),
                rdma(wr_src, wr_com.at[h], wr_send.at[h], wr_recv.at[h], left),
            ]
            for r in hops:
                r.start()
            if h == 0:
                out_ref[...] = dot32(x_ref[...], w_ref[...])
            elif h == 1:
                out_ref[TOP, LFT] += dot32(xt_com[0], wl_com[0])
                out_ref[BOT, RGT] += dot32(xb_com[0], wr_com[0])
            else:
                out_ref[TOP, LFT] += dot32(xt_com[1], wl_com[1])
                out_ref[BOT, RGT] += dot32(xb_com[1], wr_com[1])
                out_ref[TOP, RGT] += dot32(xt_com[1], wr_com[1])
                out_ref[BOT, LFT] += dot32(xb_com[1], wl_com[1])
            for r in hops:
                r.wait()

        s = scale_ref[0]
        out_ref[TOP, RGT] += dot32(xt_com[2], wr_com[0])
        out_ref[BOT, LFT] += dot32(xb_com[0], wl_com[2])
        out_ref[TOP, LFT] = (out_ref[TOP, LFT] + dot32(xt_com[2], wl_com[2])) * s
        out_ref[BOT, RGT] = (out_ref[BOT, RGT] + dot32(xb_com[2], wr_com[2])) * s
        out_ref[TOP, RGT] = (out_ref[TOP, RGT] + dot32(xt_com[0], wr_com[2])) * s
        out_ref[BOT, LFT] = (out_ref[BOT, LFT] + dot32(xb_com[2], wl_com[0])) * s

    nhops = N_DEV - 1
    return pl.pallas_call(
        body,
        out_shape=jax.ShapeDtypeStruct((m, n), jnp.float32),
        in_specs=[
            pl.BlockSpec(memory_space=pltpu.VMEM),
            pl.BlockSpec(memory_space=pltpu.VMEM),
            pl.BlockSpec(memory_space=pltpu.SMEM),
        ],
        out_specs=pl.BlockSpec(memory_space=pltpu.VMEM),
        scratch_shapes=[
            pltpu.VMEM((nhops, mh, k), jnp.float8_e4m3fn),
            pltpu.VMEM((nhops, mh, k), jnp.float8_e4m3fn),
            pltpu.VMEM((nhops, k, nh), jnp.float8_e4m3fn),
            pltpu.VMEM((nhops, k, nh), jnp.float8_e4m3fn),
            pltpu.SemaphoreType.DMA((nhops,)),
            pltpu.SemaphoreType.DMA((nhops,)),
            pltpu.SemaphoreType.DMA((nhops,)),
            pltpu.SemaphoreType.DMA((nhops,)),
            pltpu.SemaphoreType.DMA((nhops,)),
            pltpu.SemaphoreType.DMA((nhops,)),
            pltpu.SemaphoreType.DMA((nhops,)),
            pltpu.SemaphoreType.DMA((nhops,)),
        ],
        compiler_params=pltpu.CompilerParams(
            collective_id=0,
            vmem_limit_bytes=128 * 1024 * 1024,
        ),
    )(x, w_mat, scale)


# device time: 166472 ns/iter; 1.0600x vs baseline; 1.0600x over previous
import jax
import jax.numpy as jnp
from jax import lax
from jax.experimental import pallas as pl
from jax.experimental.pallas import tpu as pltpu

N_DEV = 4
F8 = jnp.float8_e4m3fn


def kernel(x, w_mat, scale_x, scale_w):
    if x.dtype != F8:
        x = x.astype(F8)
    if w_mat.dtype != F8:
        w_mat = w_mat.astype(F8)
    scale = (scale_x.astype(jnp.float32) * scale_w.astype(jnp.float32)).reshape((1,))

    m, k = x.shape
    _, n = w_mat.shape
    mh = m // 2
    nh = n // 2
    mq = mh // 2

    def body(x_ref, w_ref, scale_ref, out_ref,
             xt_com, xb_com, wl_com, wr_com,
             xt_send, xt_recv, xb_send, xb_recv,
             wl_send, wl_recv, wr_send, wr_recv):
        my = lax.axis_index("i")
        left = lax.rem(my + N_DEV - 1, N_DEV)
        right = lax.rem(my + 1, N_DEV)

        barrier_sem = pltpu.get_barrier_semaphore()
        for nbr in (left, right):
            pl.semaphore_signal(
                barrier_sem, inc=1,
                device_id=(nbr,), device_id_type=pl.DeviceIdType.MESH,
            )
        pl.semaphore_wait(barrier_sem, 2)

        def rdma(src, dst, ssem, rsem, dev):
            return pltpu.make_async_remote_copy(
                src_ref=src, dst_ref=dst, send_sem=ssem, recv_sem=rsem,
                device_id=(dev,), device_id_type=pl.DeviceIdType.MESH,
            )

        def dot32(a, b):
            return jnp.dot(a, b, preferred_element_type=jnp.float32)

        TOP, BOT = pl.ds(0, mh), pl.ds(mh, mh)
        LFT, RGT = pl.ds(0, nh), pl.ds(nh, nh)

        for h in (0, 1):
            if h == 0:
                xt_src = x_ref.at[TOP, :]
                xb_src = x_ref.at[BOT, :]
                wl_src = w_ref.at[:, LFT]
                wr_src = w_ref.at[:, RGT]
            else:
                xt_src = xt_com.at[0]
                xb_src = xb_com.at[0]
                wl_src = wl_com.at[0]
                wr_src = wr_com.at[0]
            hops = [
                rdma(xt_src, xt_com.at[h], xt_send.at[h], xt_recv.at[h], right),
                rdma(wl_src, wl_com.at[h], wl_send.at[h], wl_recv.at[h], right),
                rdma(xb_src, xb_com.at[h], xb_send.at[h], xb_recv.at[h], left),
                rdma(wr_src, wr_com.at[h], wr_send.at[h], wr_recv.at[h], left),
            ]
            for r in hops:
                r.start()
            if h == 0:
                out_ref[...] = dot32(x_ref[...], w_ref[...])
            else:
                out_ref[TOP, LFT] += dot32(xt_com[0], wl_com[0])
                out_ref[BOT, RGT] += dot32(xb_com[0], wr_com[0])
            for r in hops:
                r.wait()

        A = pl.ds(0, mq)
        B = pl.ds(mq, mq)
        sub_a = [
            rdma(xt_com.at[1, A], xt_com.at[2, A], xt_send.at[2], xt_recv.at[2], right),
            rdma(wl_com.at[1], wl_com.at[2], wl_send.at[2], wl_recv.at[2], right),
            rdma(xb_com.at[1, A], xb_com.at[2, A], xb_send.at[2], xb_recv.at[2], left),
            rdma(wr_com.at[1], wr_com.at[2], wr_send.at[2], wr_recv.at[2], left),
        ]
        sub_b = [
            rdma(xt_com.at[1, B], xt_com.at[2, B], xt_send.at[3], xt_recv.at[3], right),
            rdma(xb_com.at[1, B], xb_com.at[2, B], xb_send.at[3], xb_recv.at[3], left),
        ]
        for r in sub_a:
            r.start()
        for r in sub_b:
            r.start()

        out_ref[TOP, LFT] += dot32(xt_com[1], wl_com[1])
        out_ref[BOT, RGT] += dot32(xb_com[1], wr_com[1])
        out_ref[TOP, RGT] += dot32(xt_com[1], wr_com[1])
        out_ref[BOT, LFT] += dot32(xb_com[1], wl_com[1])

        for r in sub_a:
            r.wait()

        s = scale_ref[0]
        T_A, T_B = pl.ds(0, mq), pl.ds(mq, mq)
        B_A, B_B = pl.ds(mh, mq), pl.ds(mh + mq, mq)
        out_ref[TOP, RGT] += dot32(xt_com[0], wr_com[2])
        out_ref[BOT, LFT] += dot32(xb_com[0], wl_com[2])
        out_ref[T_A, LFT] = (out_ref[T_A, LFT] + dot32(xt_com[2, A], wl_com[2])) * s
        out_ref[T_A, RGT] = (out_ref[T_A, RGT] + dot32(xt_com[2, A], wr_com[0])) * s
        out_ref[B_A, LFT] = (out_ref[B_A, LFT] + dot32(xb_com[2, A], wl_com[0])) * s
        out_ref[B_A, RGT] = (out_ref[B_A, RGT] + dot32(xb_com[2, A], wr_com[2])) * s

        for r in sub_b:
            r.wait()

        out_ref[T_B, LFT] = (out_ref[T_B, LFT] + dot32(xt_com[2, B], wl_com[2])) * s
        out_ref[T_B, RGT] = (out_ref[T_B, RGT] + dot32(xt_com[2, B], wr_com[0])) * s
        out_ref[B_B, LFT] = (out_ref[B_B, LFT] + dot32(xb_com[2, B], wl_com[0])) * s
        out_ref[B_B, RGT] = (out_ref[B_B, RGT] + dot32(xb_com[2, B], wr_com[2])) * s

    nhops = N_DEV - 1
    return pl.pallas_call(
        body,
        out_shape=jax.ShapeDtypeStruct((m, n), jnp.float32),
        in_specs=[
            pl.BlockSpec(memory_space=pltpu.VMEM),
            pl.BlockSpec(memory_space=pltpu.VMEM),
            pl.BlockSpec(memory_space=pltpu.SMEM),
        ],
        out_specs=pl.BlockSpec(memory_space=pltpu.VMEM),
        scratch_shapes=[
            pltpu.VMEM((nhops, mh, k), F8),
            pltpu.VMEM((nhops, mh, k), F8),
            pltpu.VMEM((nhops, k, nh), F8),
            pltpu.VMEM((nhops, k, nh), F8),
            pltpu.SemaphoreType.DMA((4,)),
            pltpu.SemaphoreType.DMA((4,)),
            pltpu.SemaphoreType.DMA((4,)),
            pltpu.SemaphoreType.DMA((4,)),
            pltpu.SemaphoreType.DMA((3,)),
            pltpu.SemaphoreType.DMA((3,)),
            pltpu.SemaphoreType.DMA((3,)),
            pltpu.SemaphoreType.DMA((3,)),
        ],
        compiler_params=pltpu.CompilerParams(
            collective_id=0,
            vmem_limit_bytes=128 * 1024 * 1024,
        ),
    )(x, w_mat, scale)


# device time: 166401 ns/iter; 1.0604x vs baseline; 1.0004x over previous
import jax
import jax.numpy as jnp
from jax import lax
from jax.experimental import pallas as pl
from jax.experimental.pallas import tpu as pltpu

N_DEV = 4
F8 = jnp.float8_e4m3fn


def kernel(x, w_mat, scale_x, scale_w):
    if x.dtype != F8:
        x = x.astype(F8)
    if w_mat.dtype != F8:
        w_mat = w_mat.astype(F8)
    scale = (scale_x.astype(jnp.float32) * scale_w.astype(jnp.float32)).reshape((1,))

    m, k = x.shape
    _, n = w_mat.shape
    mh = m // 2
    nh = n // 2
    mq = mh // 2

    def body(x_ref, w_ref, scale_ref, out_ref,
             xt_com, xb_com, wL_com, wR_com, wD_com,
             xt_send, xt_recv, xb_send, xb_recv, w_send, w_recv):
        my = lax.axis_index("i")
        left = lax.rem(my + N_DEV - 1, N_DEV)
        right = lax.rem(my + 1, N_DEV)

        barrier_sem = pltpu.get_barrier_semaphore()
        for nbr in (left, right):
            pl.semaphore_signal(
                barrier_sem, inc=1,
                device_id=(nbr,), device_id_type=pl.DeviceIdType.MESH,
            )
        pl.semaphore_wait(barrier_sem, 2)

        def rdma(src, dst, ssem, rsem, dev):
            return pltpu.make_async_remote_copy(
                src_ref=src, dst_ref=dst, send_sem=ssem, recv_sem=rsem,
                device_id=(dev,), device_id_type=pl.DeviceIdType.MESH,
            )

        def dot32(a, b):
            return jnp.dot(a, b, preferred_element_type=jnp.float32)

        TOP, BOT = pl.ds(0, mh), pl.ds(mh, mh)
        LFT, RGT = pl.ds(0, nh), pl.ds(nh, nh)

        h0 = [
            rdma(x_ref.at[TOP, :], xt_com.at[0], xt_send.at[0], xt_recv.at[0], right),
            rdma(x_ref.at[BOT, :], xb_com.at[0], xb_send.at[0], xb_recv.at[0], left),
            rdma(w_ref, wL_com, w_send.at[0], w_recv.at[0], right),
            rdma(w_ref, wR_com, w_send.at[1], w_recv.at[1], left),
        ]
        for r in h0:
            r.start()
        out_ref[...] = dot32(x_ref[...], w_ref[...])
        for r in h0:
            r.wait()

        h1 = [
            rdma(xt_com.at[0], xt_com.at[1], xt_send.at[1], xt_recv.at[1], right),
            rdma(xb_com.at[0], xb_com.at[1], xb_send.at[1], xb_recv.at[1], left),
            rdma(wL_com.at[:, LFT], wD_com.at[:, LFT], w_send.at[2], w_recv.at[2], right),
            rdma(wR_com.at[:, RGT], wD_com.at[:, RGT], w_send.at[3], w_recv.at[3], left),
        ]
        for r in h1:
            r.start()
        out_ref[TOP, :] += dot32(xt_com[0], wL_com[...])
        out_ref[BOT, :] += dot32(xb_com[0], wR_com[...])
        for r in h1:
            r.wait()

        A, B = pl.ds(0, mq), pl.ds(mq, mq)
        h2a = [
            rdma(xt_com.at[1, A], xt_com.at[2, A], xt_send.at[2], xt_recv.at[2], right),
            rdma(xb_com.at[1, A], xb_com.at[2, A], xb_send.at[2], xb_recv.at[2], left),
        ]
        h2b = [
            rdma(xt_com.at[1, B], xt_com.at[2, B], xt_send.at[3], xt_recv.at[3], right),
            rdma(xb_com.at[1, B], xb_com.at[2, B], xb_send.at[3], xb_recv.at[3], left),
        ]
        for r in h2a:
            r.start()
        for r in h2b:
            r.start()
        out_ref[TOP, :] += dot32(xt_com[1], wD_com[...])
        out_ref[BOT, :] += dot32(xb_com[1], wD_com[...])
        for r in h2a:
            r.wait()

        s = scale_ref[0]
        out_ref[pl.ds(0, mq), :] = (
            out_ref[pl.ds(0, mq), :] + dot32(xt_com[2, A], wR_com[...])
        ) * s
        out_ref[pl.ds(mh, mq), :] = (
            out_ref[pl.ds(mh, mq), :] + dot32(xb_com[2, A], wL_com[...])
        ) * s

        for r in h2b:
            r.wait()

        out_ref[pl.ds(mq, mq), :] = (
            out_ref[pl.ds(mq, mq), :] + dot32(xt_com[2, B], wR_com[...])
        ) * s
        out_ref[pl.ds(mh + mq, mq), :] = (
            out_ref[pl.ds(mh + mq, mq), :] + dot32(xb_com[2, B], wL_com[...])
        ) * s

    return pl.pallas_call(
        body,
        out_shape=jax.ShapeDtypeStruct((m, n), jnp.float32),
        in_specs=[
            pl.BlockSpec(memory_space=pltpu.VMEM),
            pl.BlockSpec(memory_space=pltpu.VMEM),
            pl.BlockSpec(memory_space=pltpu.SMEM),
        ],
        out_specs=pl.BlockSpec(memory_space=pltpu.VMEM),
        scratch_shapes=[
            pltpu.VMEM((3, mh, k), F8),
            pltpu.VMEM((3, mh, k), F8),
            pltpu.VMEM((k, n), F8),
            pltpu.VMEM((k, n), F8),
            pltpu.VMEM((k, n), F8),
            pltpu.SemaphoreType.DMA((4,)),
            pltpu.SemaphoreType.DMA((4,)),
            pltpu.SemaphoreType.DMA((4,)),
            pltpu.SemaphoreType.DMA((4,)),
            pltpu.SemaphoreType.DMA((4,)),
            pltpu.SemaphoreType.DMA((4,)),
        ],
        compiler_params=pltpu.CompilerParams(
            collective_id=0,
            vmem_limit_bytes=128 * 1024 * 1024,
        ),
    )(x, w_mat, scale)


# device time: 166141 ns/iter; 1.0621x vs baseline; 1.0016x over previous
import jax
import jax.numpy as jnp
from jax import lax
from jax.experimental import pallas as pl
from jax.experimental.pallas import tpu as pltpu

N_DEV = 4
F8 = jnp.float8_e4m3fn


def _cast_to_fp8(x, w_mat):
    m, k = x.shape
    _, n = w_mat.shape
    g = 8
    bx, bw = m // g, k // g

    def body(x_ref, w_ref, x8_ref, w8_ref):
        x8_ref[...] = x_ref[...].astype(F8)
        w8_ref[...] = w_ref[...].astype(F8)

    return pl.pallas_call(
        body,
        grid=(g,),
        in_specs=[
            pl.BlockSpec((bx, k), lambda i: (i, 0)),
            pl.BlockSpec((bw, n), lambda i: (i, 0)),
        ],
        out_specs=[
            pl.BlockSpec((bx, k), lambda i: (i, 0)),
            pl.BlockSpec((bw, n), lambda i: (i, 0)),
        ],
        out_shape=(
            jax.ShapeDtypeStruct((m, k), F8),
            jax.ShapeDtypeStruct((k, n), F8),
        ),
    )(x, w_mat)


def kernel(x, w_mat, scale_x, scale_w):
    if x.dtype != F8:
        x, w_mat = _cast_to_fp8(x, w_mat)
    elif w_mat.dtype != F8:
        w_mat = w_mat.astype(F8)
    scale = (scale_x.astype(jnp.float32) * scale_w.astype(jnp.float32)).reshape((1,))

    m, k = x.shape
    _, n = w_mat.shape
    mh = m // 2
    nh = n // 2
    mq = mh // 2

    def body(x_ref, w_ref, scale_ref, out_ref,
             xt_com, xb_com, wL_com, wR_com, wD_com,
             xt_send, xt_recv, xb_send, xb_recv, w_send, w_recv):
        my = lax.axis_index("i")
        left = lax.rem(my + N_DEV - 1, N_DEV)
        right = lax.rem(my + 1, N_DEV)

        barrier_sem = pltpu.get_barrier_semaphore()
        for nbr in (left, right):
            pl.semaphore_signal(
                barrier_sem, inc=1,
                device_id=(nbr,), device_id_type=pl.DeviceIdType.MESH,
            )
        pl.semaphore_wait(barrier_sem, 2)

        def rdma(src, dst, ssem, rsem, dev):
            return pltpu.make_async_remote_copy(
                src_ref=src, dst_ref=dst, send_sem=ssem, recv_sem=rsem,
                device_id=(dev,), device_id_type=pl.DeviceIdType.MESH,
            )

        def dot32(a, b):
            return jnp.dot(a, b, preferred_element_type=jnp.float32)

        TOP, BOT = pl.ds(0, mh), pl.ds(mh, mh)
        LFT, RGT = pl.ds(0, nh), pl.ds(nh, nh)

        h0 = [
            rdma(x_ref.at[TOP, :], xt_com.at[0], xt_send.at[0], xt_recv.at[0], right),
            rdma(x_ref.at[BOT, :], xb_com.at[0], xb_send.at[0], xb_recv.at[0], left),
            rdma(w_ref, wL_com, w_send.at[0], w_recv.at[0], right),
            rdma(w_ref, wR_com, w_send.at[1], w_recv.at[1], left),
        ]
        for r in h0:
            r.start()
        out_ref[...] = dot32(x_ref[...], w_ref[...])
        for r in h0:
            r.wait()

        h1 = [
            rdma(xt_com.at[0], xt_com.at[1], xt_send.at[1], xt_recv.at[1], right),
            rdma(xb_com.at[0], xb_com.at[1], xb_send.at[1], xb_recv.at[1], left),
            rdma(wL_com.at[:, LFT], wD_com.at[:, LFT], w_send.at[2], w_recv.at[2], right),
            rdma(wR_com.at[:, RGT], wD_com.at[:, RGT], w_send.at[3], w_recv.at[3], left),
        ]
        for r in h1:
            r.start()
        out_ref[TOP, :] += dot32(xt_com[0], wL_com[...])
        out_ref[BOT, :] += dot32(xb_com[0], wR_com[...])
        for r in h1:
            r.wait()

        A, B = pl.ds(0, mq), pl.ds(mq, mq)
        h2a = [
            rdma(xt_com.at[1, A], xt_com.at[2, A], xt_send.at[2], xt_recv.at[2], right),
            rdma(xb_com.at[1, A], xb_com.at[2, A], xb_send.at[2], xb_recv.at[2], left),
        ]
        h2b = [
            rdma(xt_com.at[1, B], xt_com.at[2, B], xt_send.at[3], xt_recv.at[3], right),
            rdma(xb_com.at[1, B], xb_com.at[2, B], xb_send.at[3], xb_recv.at[3], left),
        ]
        for r in h2a:
            r.start()
        for r in h2b:
            r.start()
        out_ref[TOP, :] += dot32(xt_com[1], wD_com[...])
        out_ref[BOT, :] += dot32(xb_com[1], wD_com[...])
        for r in h2a:
            r.wait()

        s = scale_ref[0]
        out_ref[pl.ds(0, mq), :] = (
            out_ref[pl.ds(0, mq), :] + dot32(xt_com[2, A], wR_com[...])
        ) * s
        out_ref[pl.ds(mh, mq), :] = (
            out_ref[pl.ds(mh, mq), :] + dot32(xb_com[2, A], wL_com[...])
        ) * s

        for r in h2b:
            r.wait()

        out_ref[pl.ds(mq, mq), :] = (
            out_ref[pl.ds(mq, mq), :] + dot32(xt_com[2, B], wR_com[...])
        ) * s
        out_ref[pl.ds(mh + mq, mq), :] = (
            out_ref[pl.ds(mh + mq, mq), :] + dot32(xb_com[2, B], wL_com[...])
        ) * s

    return pl.pallas_call(
        body,
        out_shape=jax.ShapeDtypeStruct((m, n), jnp.float32),
        in_specs=[
            pl.BlockSpec(memory_space=pltpu.VMEM),
            pl.BlockSpec(memory_space=pltpu.VMEM),
            pl.BlockSpec(memory_space=pltpu.SMEM),
        ],
        out_specs=pl.BlockSpec(memory_space=pltpu.VMEM),
        scratch_shapes=[
            pltpu.VMEM((3, mh, k), F8),
            pltpu.VMEM((3, mh, k), F8),
            pltpu.VMEM((k, n), F8),
            pltpu.VMEM((k, n), F8),
            pltpu.VMEM((k, n), F8),
            pltpu.SemaphoreType.DMA((4,)),
            pltpu.SemaphoreType.DMA((4,)),
            pltpu.SemaphoreType.DMA((4,)),
            pltpu.SemaphoreType.DMA((4,)),
            pltpu.SemaphoreType.DMA((4,)),
            pltpu.SemaphoreType.DMA((4,)),
        ],
        compiler_params=pltpu.CompilerParams(
            collective_id=0,
            vmem_limit_bytes=128 * 1024 * 1024,
        ),
    )(x, w_mat, scale)
